# Initial kernel scaffold; baseline (speedup 1.0000x reference)
import jax
import jax.numpy as jnp
from jax.experimental import pallas as pl


def kernel(input, target):
    raise NotImplementedError("write your pallas kernel here")


# speedup vs baseline: 1.2540x; 1.2540x over previous
import functools
import math

import jax
import jax.numpy as jnp
from jax import lax
from jax.experimental import pallas as pl
from jax.experimental.pallas import tpu as pltpu
from jax.experimental.pallas import tpu_sc as plsc

B = 1024
V = 100000
M = 0.5
COS_M = math.cos(M)
SIN_M = math.sin(M)


_NC = 2
_NS = 16
_NW = _NC * _NS
_BPW = B // _NW


@functools.cache
def _build_sc_gather():
    mesh = plsc.VectorSubcoreMesh(core_axis_name="c", subcore_axis_name="s")

    @functools.partial(
        pl.kernel,
        mesh=mesh,
        out_type=jax.ShapeDtypeStruct((B,), jnp.float32),
        scratch_types=[
            pltpu.VMEM((_BPW,), jnp.int32),
            pltpu.VMEM((_BPW,), jnp.int32),
            pltpu.VMEM((_BPW,), jnp.float32),
            pltpu.SemaphoreType.DMA,
        ],
    )
    def gather_kernel(flat_hbm, tgt_hbm, out_hbm, tgt_v, idx_v, c_v, sem):
        wid = lax.axis_index("s") * _NC + lax.axis_index("c")
        base = wid * _BPW
        pltpu.sync_copy(tgt_hbm.at[pl.ds(base, _BPW)], tgt_v)
        for j in range(_BPW // 16):
            t = tgt_v[pl.ds(j * 16, 16)]
            rows = lax.iota(jnp.int32, 16) + (base + j * 16)
            idx_v[pl.ds(j * 16, 16)] = rows * V + t
        pltpu.async_copy(flat_hbm.at[idx_v], c_v, sem).wait()
        pltpu.sync_copy(c_v, out_hbm.at[pl.ds(base, _BPW)])

    return gather_kernel


_VB = 2048
_NB = -(-V // _VB)


def _tc_body(x_ref, c_ref, out_ref, acc_ref):
    j = pl.program_id(0)

    @pl.when(j == 0)
    def _init():
        acc_ref[...] = jnp.zeros_like(acc_ref)

    x = x_ref[...]
    cols = j * _VB + lax.broadcasted_iota(jnp.int32, (B, _VB), 1)
    e = jnp.where(cols < V, jnp.exp(x - 1.0), 0.0)
    acc_ref[...] += jnp.sum(e, axis=1, keepdims=True)

    @pl.when(j == _NB - 1)
    def _finish():
        c = c_ref[...]
        sin_t = jnp.sqrt(jnp.maximum(1.0 - c * c, 0.0))
        new_cos = c * COS_M - sin_t * SIN_M
        s = acc_ref[...] - jnp.exp(c - 1.0) + jnp.exp(new_cos - 1.0)
        nll = 1.0 + jnp.log(s) - new_cos
        out_ref[0, 0] = jnp.sum(nll) / B


def _tc_loss(inp, c):
    return pl.pallas_call(
        _tc_body,
        grid=(_NB,),
        in_specs=[
            pl.BlockSpec((B, _VB), lambda j: (0, j)),
            pl.BlockSpec((B, 1), lambda j: (0, 0)),
        ],
        out_specs=pl.BlockSpec(memory_space=pltpu.SMEM),
        out_shape=jax.ShapeDtypeStruct((1, 1), jnp.float32),
        scratch_shapes=[pltpu.VMEM((B, 1), jnp.float32)],
    )(inp, c)


def kernel(input, target):
    flat = input.reshape(B * V)
    c = _build_sc_gather()(flat, target.astype(jnp.int32))
    out = _tc_loss(input, c.reshape(B, 1))
    return out[0, 0]


# speedup vs baseline: 1.2576x; 1.0029x over previous
import functools
import math

import jax
import jax.numpy as jnp
from jax import lax
from jax.experimental import pallas as pl
from jax.experimental.pallas import tpu as pltpu
from jax.experimental.pallas import tpu_sc as plsc

B = 1024
V = 100000
M = 0.5
COS_M = math.cos(M)
SIN_M = math.sin(M)


_NC = 2
_NS = 16
_NW = _NC * _NS
_BPW = B // _NW


@functools.cache
def _build_sc_gather():
    mesh = plsc.VectorSubcoreMesh(core_axis_name="c", subcore_axis_name="s")

    @functools.partial(
        pl.kernel,
        mesh=mesh,
        out_type=jax.ShapeDtypeStruct((B,), jnp.float32),
        scratch_types=[
            pltpu.VMEM((_BPW,), jnp.int32),
            pltpu.VMEM((_BPW,), jnp.int32),
            pltpu.VMEM((_BPW,), jnp.float32),
            pltpu.SemaphoreType.DMA,
        ],
    )
    def gather_kernel(flat_hbm, tgt_hbm, out_hbm, tgt_v, idx_v, c_v, sem):
        wid = lax.axis_index("s") * _NC + lax.axis_index("c")
        base = wid * _BPW
        pltpu.sync_copy(tgt_hbm.at[pl.ds(base, _BPW)], tgt_v)
        for j in range(_BPW // 16):
            t = tgt_v[pl.ds(j * 16, 16)]
            rows = lax.iota(jnp.int32, 16) + (base + j * 16)
            idx_v[pl.ds(j * 16, 16)] = rows * V + t
        pltpu.async_copy(flat_hbm.at[idx_v], c_v, sem).wait()
        pltpu.sync_copy(c_v, out_hbm.at[pl.ds(base, _BPW)])

    return gather_kernel


_VB = 2048
_NB = -(-V // _VB)


def _tc_body(x_ref, c_ref, out_ref, acc_ref):
    j = pl.program_id(0)

    @pl.when(j == 0)
    def _init():
        acc_ref[...] = jnp.zeros_like(acc_ref)

    @pl.when(j < _NB - 1)
    def _full_block():
        x = x_ref[...]
        s = jnp.exp(x[:, 0:128])
        for k in range(1, _VB // 128):
            s += jnp.exp(x[:, k * 128:(k + 1) * 128])
        acc_ref[...] += s

    @pl.when(j == _NB - 1)
    def _last_block():
        x = x_ref[...]
        cols = j * _VB + lax.broadcasted_iota(jnp.int32, (B, _VB), 1)
        e = jnp.where(cols < V, jnp.exp(x), 0.0)
        s = e[:, 0:128]
        for k in range(1, _VB // 128):
            s += e[:, k * 128:(k + 1) * 128]
        acc = acc_ref[...] + s
        rowsum = jnp.sum(acc, axis=1, keepdims=True)
        c = c_ref[...]
        sin_t = jnp.sqrt(jnp.maximum(1.0 - c * c, 0.0))
        new_cos = c * COS_M - sin_t * SIN_M
        stot = rowsum - jnp.exp(c) + jnp.exp(new_cos)
        nll = jnp.log(stot) - new_cos
        out_ref[0, 0] = jnp.sum(nll) / B


def _tc_loss(inp, c):
    return pl.pallas_call(
        _tc_body,
        grid=(_NB,),
        in_specs=[
            pl.BlockSpec((B, _VB), lambda j: (0, j)),
            pl.BlockSpec((B, 1), lambda j: (0, 0)),
        ],
        out_specs=pl.BlockSpec(memory_space=pltpu.SMEM),
        out_shape=jax.ShapeDtypeStruct((1, 1), jnp.float32),
        scratch_shapes=[pltpu.VMEM((B, 128), jnp.float32)],
    )(inp, c)


def kernel(input, target):
    flat = input.reshape(B * V)
    c = _build_sc_gather()(flat, target.astype(jnp.int32))
    out = _tc_loss(input, c.reshape(B, 1))
    return out[0, 0]
